# baseline (speedup 1.0000x reference)
import functools
import math

import jax
import jax.numpy as jnp
from jax import lax
from jax.experimental import pallas as pl
from jax.experimental.pallas import tpu as pltpu
from jax.experimental.pallas import tpu_sc as plsc

NLAT_IN, NLON_IN = 181, 360
NLAT_OUT, NLON_OUT = 91, 180
KSIZE = 5
C = 64
NL = 7
ROWS = 2 * NL
MAXE = 3080
CBLK = 4
NTASK = NLAT_OUT * CBLK
NWORK = 32
TPW = -(-NTASK // NWORK)


def _sc_stage(xpd, mrow, val, offs, task_t, task_cb, lat0):
    mesh = plsc.VectorSubcoreMesh(core_axis_name="c", subcore_axis_name="s")

    @functools.partial(
        pl.kernel,
        out_type=jax.ShapeDtypeStruct((NLAT_OUT, NLON_OUT, KSIZE * C), jnp.float32),
        mesh=mesh,
        compiler_params=pltpu.CompilerParams(use_tc_tiling_on_sc=False),
        scratch_types=[
            pltpu.VMEM((NL, 720, 16), jnp.float32),
            pltpu.VMEM((KSIZE * 180, 16), jnp.float32),
            pltpu.VMEM((MAXE,), jnp.int32),
            pltpu.VMEM((MAXE,), jnp.float32),
            pltpu.VMEM((456 + 16,), jnp.int32),
            pltpu.VMEM((NWORK * TPW + 16,), jnp.int32),
            pltpu.VMEM((NWORK * TPW + 16,), jnp.int32),
            pltpu.VMEM((NLAT_OUT + 16,), jnp.int32),
            pltpu.SemaphoreType.DMA,
            pltpu.SemaphoreType.DMA,
        ],
    )
    def kern(xpd_hbm, mrow_hbm, val_hbm, offs_hbm, taskt_hbm, taskcb_hbm,
             lat0_hbm, y_hbm, band, acc, mr, mv, offs_v, taskt_v, taskcb_v,
             lat0_v, sem, sem_out):
        w = lax.axis_index("s") * 2 + lax.axis_index("c")
        pltpu.sync_copy(offs_hbm, offs_v)
        pltpu.sync_copy(taskt_hbm, taskt_v)
        pltpu.sync_copy(taskcb_hbm, taskcb_v)
        pltpu.sync_copy(lat0_hbm, lat0_v)
        zero16 = jnp.zeros((16,), jnp.float32)

        def sload(ref, i):
            return ref[pl.ds(i, 16)][0]

        @pl.loop(0, TPW)
        def _task(jj):
            m = w + jj * NWORK
            tt = sload(taskt_v, m)

            @pl.when(tt >= 0)
            def _():
                cb = sload(taskcb_v, m)
                l0 = sload(lat0_v, tt)
                base = sload(offs_v, tt * KSIZE)
                base8 = (base // 8) * 8
                src = xpd_hbm.at[pl.ds(l0, NL), :, pl.ds(cb * 16, 16)]
                cp1a = pltpu.async_copy(src, band.at[:, 0:360, :], sem)
                cp1b = pltpu.async_copy(src, band.at[:, 360:720, :], sem)
                cp2 = pltpu.async_copy(mrow_hbm.at[pl.ds(base8, MAXE)], mr, sem)
                cp3 = pltpu.async_copy(val_hbm.at[pl.ds(base8, MAXE)], mv, sem)
                cp1a.wait()
                cp1b.wait()
                cp2.wait()
                cp3.wait()

                PT = 18
                for kk in range(KSIZE):
                    e0 = sload(offs_v, tt * KSIZE + kk) - base8
                    e1 = sload(offs_v, tt * KSIZE + kk + 1) - base8
                    npair = (e1 - e0 + 1) // 2
                    for pt in range(180 // PT):
                        pbase = pt * PT

                        def ebody(ip, accs, pbase=pbase, e0=e0, e1=e1):
                            e = e0 + 2 * ip
                            m1 = sload(mr, e)
                            r1 = lax.shift_right_logical(m1, 10)
                            u1 = (m1 & 1023) + 2 * pbase
                            v1 = sload(mv, e)
                            m2 = sload(mr, e + 1)
                            r2 = lax.shift_right_logical(m2, 10)
                            u2 = (m2 & 1023) + 2 * pbase
                            v2 = jnp.where(e + 1 < e1, sload(mv, e + 1), 0.0)
                            return tuple(
                                a + v1 * band[r1, u1 + 2 * i]
                                + v2 * band[r2, u2 + 2 * i]
                                for i, a in enumerate(accs))

                        accs = lax.fori_loop(
                            0, npair, ebody,
                            tuple(zero16 for _ in range(PT)))
                        for i in range(PT):
                            acc[kk * 180 + pbase + i] = accs[i]

                outs = []
                for kk in range(KSIZE):
                    outs.append(pltpu.async_copy(
                        acc.at[pl.ds(kk * 180, 180)],
                        y_hbm.at[tt, :, pl.ds(kk * C + cb * 16, 16)],
                        sem_out))
                for cp in outs:
                    cp.wait()

    return kern(xpd, mrow, val, offs, task_t, task_cb, lat0)


def _xpd_stage(x, quad_weights):
    LB = 8
    nblk = -(-NLAT_IN // LB)

    def body(x_ref, q_ref, o_ref):
        xb = x_ref[0].reshape(C, LB * 360)
        y = jnp.transpose(xb).reshape(LB, 360, C)
        o_ref[...] = y * q_ref[...][:, :, None]

    return pl.pallas_call(
        body,
        grid=(nblk,),
        in_specs=[pl.BlockSpec((1, C, LB, 360), lambda l: (0, 0, l, 0)),
                  pl.BlockSpec((LB, 1), lambda l: (l, 0))],
        out_specs=pl.BlockSpec((LB, 360, C), lambda l: (l, 0, 0)),
        out_shape=jax.ShapeDtypeStruct((NLAT_IN, 360, C), jnp.float32),
    )(x, quad_weights)


def _tc_stage(y, w2, bias2):
    TB = 8

    def body(y_ref, w_ref, b_ref, o_ref):
        yb = y_ref[...].reshape(TB * NLON_OUT, KSIZE * C)
        res = lax.dot_general(
            w_ref[...], yb, (((1,), (1,)), ((), ())),
            preferred_element_type=jnp.float32,
            precision=lax.Precision.HIGHEST)
        o_ref[0] = res.reshape(C, TB, NLON_OUT) + b_ref[...][:, :, None]

    return pl.pallas_call(
        body,
        grid=(-(-NLAT_OUT // TB),),
        in_specs=[
            pl.BlockSpec((TB, NLON_OUT, KSIZE * C), lambda t: (t, 0, 0)),
            pl.BlockSpec((C, KSIZE * C), lambda t: (0, 0)),
            pl.BlockSpec((C, 1), lambda t: (0, 0)),
        ],
        out_specs=pl.BlockSpec((1, C, TB, NLON_OUT), lambda t: (0, 0, t, 0)),
        out_shape=jax.ShapeDtypeStruct((1, C, NLAT_OUT, NLON_OUT), jnp.float32),
    )(y, w2, bias2)


def kernel(x, weight, bias, quad_weights, psi_idx, psi_vals):
    xpd = _xpd_stage(x, quad_weights)

    k_j = psi_idx[0].astype(jnp.int32)
    t_j = psi_idx[1].astype(jnp.int32)
    flat = psi_idx[2].astype(jnp.int32)
    lat = flat // NLON_IN
    lon = flat % NLON_IN

    def band_lat0(tv):
        return jnp.clip(2 * tv - 3, 0, NLAT_IN - NL)

    lat0 = band_lat0(jnp.arange(NLAT_OUT)).astype(jnp.int32)

    key = t_j * KSIZE + k_j
    offs = jnp.sum(
        key[None, :] < jnp.arange(KSIZE * NLAT_OUT + 1, dtype=jnp.int32)[:, None],
        axis=1, dtype=jnp.int32)
    offs_pad = jnp.concatenate([offs, jnp.zeros((16,), jnp.int32)])

    tidx = jnp.arange(NTASK, dtype=jnp.int32) // CBLK
    order_t = jnp.where(tidx % 2 == 0, (NLAT_OUT - 1) - tidx // 2, tidx // 2)
    task_t = order_t.astype(jnp.int32)
    task_cb = (jnp.arange(NTASK, dtype=jnp.int32) % CBLK)
    pad_n = NWORK * TPW - NTASK
    task_t = jnp.concatenate([task_t, -jnp.ones((pad_n + 16,), jnp.int32)])
    task_cb = jnp.concatenate([task_cb, jnp.zeros((pad_n + 16,), jnp.int32)])
    lat0_pad = jnp.concatenate([lat0, jnp.zeros((16,), jnp.int32)])

    zpad_i = jnp.zeros((MAXE,), jnp.int32)
    zpad_f = jnp.zeros((MAXE,), jnp.float32)
    mrow = ((lat - band_lat0(t_j)) * 1024 + lon).astype(jnp.int32)
    mrow_p = jnp.concatenate([mrow, zpad_i])
    val_p = jnp.concatenate([psi_vals, zpad_f])

    y = _sc_stage(xpd, mrow_p, val_p, offs_pad, task_t, task_cb, lat0_pad)

    w2 = weight.transpose(0, 2, 1).reshape(C, KSIZE * C)
    bias2 = bias.reshape(C, 1)
    return _tc_stage(y, w2, bias2)


# speedup vs baseline: 1.0076x; 1.0076x over previous
import functools

import jax
import jax.numpy as jnp
from jax import lax
from jax.experimental import pallas as pl
from jax.experimental.pallas import tpu as pltpu
from jax.experimental.pallas import tpu_sc as plsc

NLAT_IN, NLON_IN = 181, 360
NLAT_OUT, NLON_OUT = 91, 180
KSIZE = 5
C = 64
NL = 7
MAXE = 3080
CBLK = 4
NTASK = NLAT_OUT * CBLK
NWORK = 32
TPW = -(-NTASK // NWORK)


def _sc_stage(xpd, mrow, val, offs, task_t, task_cb, lat0):
    mesh = plsc.VectorSubcoreMesh(core_axis_name="c", subcore_axis_name="s")

    @functools.partial(
        pl.kernel,
        out_type=jax.ShapeDtypeStruct((NLAT_OUT, NLON_OUT, KSIZE * C), jnp.float32),
        mesh=mesh,
        compiler_params=pltpu.CompilerParams(use_tc_tiling_on_sc=False),
        scratch_types=[
            pltpu.VMEM((NL * 720, 16), jnp.float32),
            pltpu.VMEM((KSIZE * 180, 16), jnp.float32),
            pltpu.VMEM((MAXE,), jnp.int32),
            pltpu.VMEM((MAXE,), jnp.float32),
            pltpu.VMEM((456 + 16,), jnp.int32),
            pltpu.VMEM((NWORK * TPW + 16,), jnp.int32),
            pltpu.VMEM((NWORK * TPW + 16,), jnp.int32),
            pltpu.VMEM((NLAT_OUT + 16,), jnp.int32),
            pltpu.SemaphoreType.DMA,
            pltpu.SemaphoreType.DMA,
        ],
    )
    def kern(xpd_hbm, mrow_hbm, val_hbm, offs_hbm, taskt_hbm, taskcb_hbm,
             lat0_hbm, y_hbm, band, acc, mr, mv, offs_v, taskt_v, taskcb_v,
             lat0_v, sem, sem_out):
        w = lax.axis_index("s") * 2 + lax.axis_index("c")
        pltpu.sync_copy(offs_hbm, offs_v)
        pltpu.sync_copy(taskt_hbm, taskt_v)
        pltpu.sync_copy(taskcb_hbm, taskcb_v)
        pltpu.sync_copy(lat0_hbm, lat0_v)
        zero16 = jnp.zeros((16,), jnp.float32)

        def sload(ref, i):
            return ref[pl.ds(i, 16)][0]

        @pl.loop(0, TPW)
        def _task(jj):
            m = w + jj * NWORK
            tt = sload(taskt_v, m)

            @pl.when(tt >= 0)
            def _():
                cb = sload(taskcb_v, m)
                l0 = sload(lat0_v, tt)
                base = sload(offs_v, tt * KSIZE)
                base8 = (base // 8) * 8
                cp1 = pltpu.async_copy(
                    xpd_hbm.at[pl.ds(l0 * 720, NL * 720),
                               pl.ds(cb * 16, 16)],
                    band, sem)
                cp2 = pltpu.async_copy(mrow_hbm.at[pl.ds(base8, MAXE)], mr, sem)
                cp3 = pltpu.async_copy(val_hbm.at[pl.ds(base8, MAXE)], mv, sem)
                cp1.wait()
                cp2.wait()
                cp3.wait()

                PT = 15
                for kk in range(KSIZE):
                    e0 = sload(offs_v, tt * KSIZE + kk) - base8
                    e1 = sload(offs_v, tt * KSIZE + kk + 1) - base8
                    for pt in range(180 // PT):
                        pbase = pt * PT

                        def ebody(e, accs, pbase=pbase):
                            rb = sload(mr, e) + 2 * pbase
                            vv = sload(mv, e)
                            return tuple(
                                a + vv * band[rb + 2 * i]
                                for i, a in enumerate(accs))

                        accs = lax.fori_loop(
                            e0, e1, ebody,
                            tuple(zero16 for _ in range(PT)))
                        for i in range(PT):
                            acc[kk * 180 + pbase + i] = accs[i]

                outs = []
                for kk in range(KSIZE):
                    outs.append(pltpu.async_copy(
                        acc.at[pl.ds(kk * 180, 180)],
                        y_hbm.at[tt, :, pl.ds(kk * C + cb * 16, 16)],
                        sem_out))
                for cp in outs:
                    cp.wait()

    return kern(xpd, mrow, val, offs, task_t, task_cb, lat0)


def _xpd_stage(x, quad_weights):
    LB = 8
    nblk = -(-NLAT_IN // LB)

    def body(x_ref, q_ref, o_ref):
        xb = x_ref[0].reshape(C, LB * 360)
        y = jnp.transpose(xb).reshape(LB, 360, C)
        y = y * q_ref[...][:, :, None]
        o_ref[...] = jnp.concatenate([y, y], axis=1).reshape(LB * 720, C)

    return pl.pallas_call(
        body,
        grid=(nblk,),
        in_specs=[pl.BlockSpec((1, C, LB, 360), lambda l: (0, 0, l, 0)),
                  pl.BlockSpec((LB, 1), lambda l: (l, 0))],
        out_specs=pl.BlockSpec((LB * 720, C), lambda l: (l, 0)),
        out_shape=jax.ShapeDtypeStruct((NLAT_IN * 720, C), jnp.float32),
    )(x, quad_weights)


def _tc_stage(y, w2, bias2):
    TB = 8

    def body(y_ref, w_ref, b_ref, o_ref):
        yb = y_ref[...].reshape(TB * NLON_OUT, KSIZE * C)
        res = lax.dot_general(
            w_ref[...], yb, (((1,), (1,)), ((), ())),
            preferred_element_type=jnp.float32,
            precision=lax.Precision.HIGHEST)
        o_ref[0] = res.reshape(C, TB, NLON_OUT) + b_ref[...][:, :, None]

    return pl.pallas_call(
        body,
        grid=(-(-NLAT_OUT // TB),),
        in_specs=[
            pl.BlockSpec((TB, NLON_OUT, KSIZE * C), lambda t: (t, 0, 0)),
            pl.BlockSpec((C, KSIZE * C), lambda t: (0, 0)),
            pl.BlockSpec((C, 1), lambda t: (0, 0)),
        ],
        out_specs=pl.BlockSpec((1, C, TB, NLON_OUT), lambda t: (0, 0, t, 0)),
        out_shape=jax.ShapeDtypeStruct((1, C, NLAT_OUT, NLON_OUT), jnp.float32),
    )(y, w2, bias2)


def kernel(x, weight, bias, quad_weights, psi_idx, psi_vals):
    xpd = _xpd_stage(x, quad_weights)

    k_j = psi_idx[0].astype(jnp.int32)
    t_j = psi_idx[1].astype(jnp.int32)
    flat = psi_idx[2].astype(jnp.int32)
    lat = flat // NLON_IN
    lon = flat % NLON_IN

    def band_lat0(tv):
        return jnp.clip(2 * tv - 3, 0, NLAT_IN - NL)

    lat0 = band_lat0(jnp.arange(NLAT_OUT)).astype(jnp.int32)

    key = t_j * KSIZE + k_j
    offs = jnp.sum(
        key[None, :] < jnp.arange(KSIZE * NLAT_OUT + 1, dtype=jnp.int32)[:, None],
        axis=1, dtype=jnp.int32)
    offs_pad = jnp.concatenate([offs, jnp.zeros((16,), jnp.int32)])

    tidx = jnp.arange(NTASK, dtype=jnp.int32) // CBLK
    order_t = jnp.where(tidx % 2 == 0, (NLAT_OUT - 1) - tidx // 2, tidx // 2)
    task_t = order_t.astype(jnp.int32)
    task_cb = (jnp.arange(NTASK, dtype=jnp.int32) % CBLK)
    pad_n = NWORK * TPW - NTASK
    task_t = jnp.concatenate([task_t, -jnp.ones((pad_n + 16,), jnp.int32)])
    task_cb = jnp.concatenate([task_cb, jnp.zeros((pad_n + 16,), jnp.int32)])
    lat0_pad = jnp.concatenate([lat0, jnp.zeros((16,), jnp.int32)])

    zpad_i = jnp.zeros((MAXE,), jnp.int32)
    zpad_f = jnp.zeros((MAXE,), jnp.float32)
    mrow = ((lat - band_lat0(t_j)) * 720 + lon).astype(jnp.int32)
    mrow_p = jnp.concatenate([mrow, zpad_i])
    val_p = jnp.concatenate([psi_vals, zpad_f])

    y = _sc_stage(xpd, mrow_p, val_p, offs_pad, task_t, task_cb, lat0_pad)

    w2 = weight.transpose(0, 2, 1).reshape(C, KSIZE * C)
    bias2 = bias.reshape(C, 1)
    return _tc_stage(y, w2, bias2)
